# Initial kernel scaffold; baseline (speedup 1.0000x reference)
import jax
import jax.numpy as jnp
from jax.experimental import pallas as pl


def kernel(x, W, b, W2, b2):
    raise NotImplementedError("write your pallas kernel here")


# speedup vs baseline: 2.9146x; 2.9146x over previous
import math

import jax
import jax.numpy as jnp
from jax.experimental import pallas as pl
from jax.experimental.pallas import tpu as pltpu

D_MODEL = 128
PATCH_LEN = 32
STRIDE = 16
KEEP = D_MODEL - D_MODEL // 4
DPOS = D_MODEL // 4
_SIN_PI = math.sin(math.pi)
_COS_PI = math.cos(math.pi)


def _body(z_ref, wab_ref, b96_ref, w2t_ref, b2_ref,
          comb_ref, pos_ref, len_ref):
    B, P = pos_ref.shape
    R, K2 = z_ref.shape
    z = z_ref[...]
    wab = wab_ref[...]
    m = jax.lax.dot_general(z, wab, (((1,), (0,)), ((), ())),
                            preferred_element_type=jnp.float32)
    a = m[:, :KEEP]
    bpart = m[:, KEEP:]
    bshift = jnp.concatenate([bpart[1:, :], jnp.zeros((1, KEEP), jnp.float32)],
                             axis=0)
    out96 = a + bshift + b96_ref[0, :][None, :]
    out96 = out96.reshape(B, R // B, KEEP)[:, :P, :]
    pos_emb = (w2t_ref[0:1, :] * _SIN_PI + w2t_ref[1:2, :] * _COS_PI
               + b2_ref[0:1, :])
    pe = jnp.broadcast_to(pos_emb[None, :, :], (B, P, DPOS))
    comb_ref[...] = jnp.concatenate([out96, pe], axis=2)
    pidx = jax.lax.broadcasted_iota(jnp.int32, (B, P), 1)
    pos_ref[...] = pidx.astype(jnp.float32) * float(STRIDE) + PATCH_LEN // 2
    len_ref[...] = jnp.full((B, P), float(PATCH_LEN), jnp.float32)


def kernel(x, W, b, W2, b2):
    B, S, V = x.shape
    n_patches = max(1, (S - PATCH_LEN) // STRIDE + 1)
    nblk = S // STRIDE
    z = x.reshape(B * nblk, STRIDE * V)
    wflat = jnp.repeat(W[:KEEP] * (1.0 / V), V, axis=1)
    wab = jnp.concatenate([wflat[:, :STRIDE * V].T,
                           wflat[:, STRIDE * V:].T], axis=1)
    b96 = b[:KEEP].reshape(1, KEEP)
    w2t = W2.T
    b2r = b2.reshape(1, DPOS)

    comb, pos, plen = pl.pallas_call(
        _body,
        out_shape=(
            jax.ShapeDtypeStruct((B, n_patches, D_MODEL), jnp.float32),
            jax.ShapeDtypeStruct((B, n_patches), jnp.float32),
            jax.ShapeDtypeStruct((B, n_patches), jnp.float32),
        ),
    )(z, wab, b96, w2t, b2r)
    return comb, pos, plen
